# baseline (speedup 1.0000x reference)
import functools

import jax
import jax.numpy as jnp
from jax import lax
from jax.experimental import pallas as pl
from jax.experimental.pallas import tpu as pltpu
from jax.experimental.pallas import tpu_sc as plsc

_D = 32


@functools.lru_cache(maxsize=None)
def _make_gather(B):
    info = plsc.get_sparse_core_info()
    NC, NS, L = info.num_cores, info.num_subcores, info.num_lanes
    NW = NC * NS
    assert B % (8 * NW) == 0
    b_per_w = B // NW
    C = 256
    while b_per_w % (2 * C):
        C //= 2
    n_chunks = b_per_w // C
    mesh = plsc.VectorSubcoreMesh(core_axis_name="c", subcore_axis_name="s")

    @functools.partial(
        pl.kernel,
        mesh=mesh,
        out_type=jax.ShapeDtypeStruct((B * _D,), jnp.float32),
        scratch_types=[
            pltpu.VMEM((b_per_w,), jnp.int32),
            pltpu.VMEM((b_per_w,), jnp.int32),
            pltpu.VMEM((C, 4 * _D), jnp.float32),
            pltpu.VMEM((C, 4 * _D), jnp.float32),
            pltpu.VMEM((C * _D,), jnp.float32),
            pltpu.VMEM((C * _D,), jnp.float32),
            pltpu.SemaphoreType.DMA,
            pltpu.SemaphoreType.DMA,
        ],
        compiler_params=pltpu.CompilerParams(needs_layout_passes=False),
    )
    def k(tab_hbm, idx_hbm, out_hbm, idx4_v, col_v, grp0_v, grp1_v, row0_v,
          row1_v, sem_g, sem_o):
        grp_b = (grp0_v, grp1_v)
        row_b = (row0_v, row1_v)
        wid = lax.axis_index("s") * NC + lax.axis_index("c")
        base = wid * b_per_w
        pltpu.sync_copy(idx_hbm.at[pl.ds(base, b_per_w)], idx4_v)

        iota = lax.iota(jnp.int32, L)

        def prep(s, _):
            v = idx4_v[pl.ds(s * L, L)]
            idx4_v[pl.ds(s * L, L)] = lax.shift_right_logical(v, 2)
            col_v[pl.ds(s * L, L)] = (v & 3) * _D
            return 0

        lax.fori_loop(0, b_per_w // L, prep, 0)

        def gather(g, b):
            return pltpu.async_copy(
                tab_hbm.at[idx4_v.at[pl.ds(g * C, C)]], grp_b[b], sem_g
            )

        def put(g, b):
            return pltpu.async_copy(
                row_b[b],
                out_hbm.at[pl.ds((base + g * C) * _D, C * _D)],
                sem_o,
            )

        def extract(g, b):
            def body(t, _):
                row = t * L + iota
                row32 = row * _D
                col = col_v[pl.ds(g * C + t * L, L)]
                for kk in range(_D):
                    x = plsc.load_gather(grp_b[b], [row, col + kk])
                    plsc.store_scatter(row_b[b], [row32 + kk], x)
                return 0

            lax.fori_loop(0, C // L, body, 0)

        gather(0, 0)

        def chunk_pair(c, _):
            for b in range(2):
                g = c * 2 + b

                @pl.when(g + 1 < n_chunks)
                def _():
                    gather(g + 1, 1 - b)

                pltpu.make_async_copy(
                    tab_hbm.at[idx4_v.at[pl.ds(g * C, C)]], grp_b[b], sem_g
                ).wait()

                @pl.when(g >= 2)
                def _():
                    pltpu.make_async_copy(
                        row_b[b],
                        out_hbm.at[pl.ds(base * _D, C * _D)],
                        sem_o,
                    ).wait()

                extract(g, b)
                put(g, b)
            return 0

        lax.fori_loop(0, n_chunks // 2, chunk_pair, 0)
        for b in range(2):
            pltpu.make_async_copy(
                row_b[b],
                out_hbm.at[pl.ds(base * _D, C * _D)],
                sem_o,
            ).wait()

    return k


def kernel(token_ids, weight):
    B = token_ids.shape[0] * token_ids.shape[1]
    V, D = weight.shape
    flat = token_ids.reshape(B).astype(jnp.int32)
    w4 = weight.reshape(V // 4, 4 * D)
    out = _make_gather(B)(w4, flat)
    return out.reshape(token_ids.shape + (D,))


# speedup vs baseline: 2.0217x; 2.0217x over previous
import functools

import jax
import jax.numpy as jnp
from jax import lax
from jax.experimental import pallas as pl
from jax.experimental.pallas import tpu as pltpu
from jax.experimental.pallas import tpu_sc as plsc

_D = 32
_RB = 128


@functools.lru_cache(maxsize=None)
def _make_gather(R, J):
    info = plsc.get_sparse_core_info()
    NC, NS, L = info.num_cores, info.num_subcores, info.num_lanes
    NW = NC * NS
    assert R % (_RB * NW) == 0
    SPW = R // (_RB * NW)
    n_cells = SPW * J
    assert n_cells % 2 == 0
    b_per_w = SPW * _RB * J
    B = R * J
    mesh = plsc.VectorSubcoreMesh(core_axis_name="c", subcore_axis_name="s")

    @functools.partial(
        pl.kernel,
        mesh=mesh,
        out_type=jax.ShapeDtypeStruct((B * _D,), jnp.float32),
        scratch_types=[
            pltpu.VMEM((b_per_w,), jnp.int32),
            pltpu.VMEM((b_per_w,), jnp.int32),
            pltpu.VMEM((_RB, _D), jnp.float32),
            pltpu.VMEM((_RB, _D), jnp.float32),
            pltpu.VMEM((_RB * _D,), jnp.float32),
            pltpu.VMEM((_RB * _D,), jnp.float32),
            pltpu.SemaphoreType.DMA,
            pltpu.SemaphoreType.DMA,
        ],
        compiler_params=pltpu.CompilerParams(
            use_tc_tiling_on_sc=False, needs_layout_passes=False
        ),
    )
    def k(tab_hbm, idx_hbm, out_hbm, raw_v, col_v, grp0_v, grp1_v, ob0_v,
          ob1_v, sem_g, sem_o):
        grp_b = (grp0_v, grp1_v)
        ob_b = (ob0_v, ob1_v)
        wid = lax.axis_index("s") * NC + lax.axis_index("c")
        base = wid * b_per_w
        pltpu.sync_copy(idx_hbm.at[pl.ds(base, b_per_w)], raw_v)

        iota = lax.iota(jnp.int32, L)
        iotaJ = iota * J

        def regroup(g, _):
            sl, j = lax.div(g, J), lax.rem(g, J)
            for m in range(_RB // L):
                src = sl * (_RB * J) + (m * L) * J + j + iotaJ
                x = plsc.load_gather(raw_v, [src])
                col_v[pl.ds(g * _RB + m * L, L)] = x
            return 0

        lax.fori_loop(0, n_cells, regroup, 0)

        def gather(t, b):
            return pltpu.async_copy(
                tab_hbm.at[col_v.at[pl.ds(t * _RB, _RB)]], grp_b[b], sem_g
            )

        sks = [(iota + m) & (_D - 1) for m in range(_D)]
        dks = [
            lax.shift_right_logical(sk, 3) * (8 * _RB)
            + (sk & 7) * _RB
            + iota
            for sk in sks
        ]

        def transpose(b):
            def body(q, _):
                b0 = q * L
                for m in range(_D):
                    x = plsc.load_gather(grp_b[b], [b0 + iota, sks[m]])
                    plsc.store_scatter(ob_b[b], [dks[m] + b0], x)
                return 0

            lax.fori_loop(0, _RB // L, body, 0)

        def put(t, b):
            sl, j = lax.div(t, J), lax.rem(t, J)
            s = wid * SPW + sl
            for fb in range(_D // 8):
                pltpu.async_copy(
                    ob_b[b].at[pl.ds(fb * (8 * _RB), 8 * _RB)],
                    out_hbm.at[
                        pl.ds(((j * (_D // 8) + fb) * (R // _RB) + s)
                              * (8 * _RB), 8 * _RB)
                    ],
                    sem_o,
                )

        def wait_puts(b):
            pltpu.make_async_copy(
                ob_b[b], out_hbm.at[pl.ds(0, _RB * _D)], sem_o
            ).wait()

        gather(0, 0)

        def cell_pair(c, _):
            for b in range(2):
                t = c * 2 + b

                @pl.when(t + 1 < n_cells)
                def _():
                    gather(t + 1, 1 - b)

                pltpu.make_async_copy(
                    tab_hbm.at[col_v.at[pl.ds(t * _RB, _RB)]], grp_b[b], sem_g
                ).wait()

                @pl.when(t >= 2)
                def _():
                    wait_puts(b)

                transpose(b)
                put(t, b)
            return 0

        lax.fori_loop(0, n_cells // 2, cell_pair, 0)
        for b in range(2):
            wait_puts(b)

    return k


def kernel(token_ids, weight):
    R, J = token_ids.shape
    V, D = weight.shape
    flat = token_ids.reshape(R * J).astype(jnp.int32)
    out = _make_gather(R, J)(weight, flat)
    o5 = out.reshape(J, D // 8, R // _RB, 8, _RB)
    return o5.transpose(2, 4, 0, 1, 3).reshape(R, J, D)


# speedup vs baseline: 2.1319x; 1.0545x over previous
import functools

import jax
import jax.numpy as jnp
from jax import lax
from jax.experimental import pallas as pl
from jax.experimental.pallas import tpu as pltpu
from jax.experimental.pallas import tpu_sc as plsc

_D = 32
_RB = 128


@functools.lru_cache(maxsize=None)
def _make_gather(R, J):
    info = plsc.get_sparse_core_info()
    NC, NS, L = info.num_cores, info.num_subcores, info.num_lanes
    NW = NC * NS
    assert R % (_RB * NW) == 0
    SPW = R // (_RB * NW)
    n_cells = SPW * J
    assert n_cells % 2 == 0
    b_per_w = SPW * _RB * J
    B = R * J
    mesh = plsc.VectorSubcoreMesh(core_axis_name="c", subcore_axis_name="s")

    @functools.partial(
        pl.kernel,
        mesh=mesh,
        out_type=jax.ShapeDtypeStruct((B * _D,), jnp.float32),
        scratch_types=[
            pltpu.VMEM((b_per_w,), jnp.int32),
            pltpu.VMEM((b_per_w,), jnp.int32),
            pltpu.VMEM((_RB, _D), jnp.float32),
            pltpu.VMEM((_RB, _D), jnp.float32),
            pltpu.VMEM((_RB * _D,), jnp.float32),
            pltpu.VMEM((_RB * _D,), jnp.float32),
            pltpu.SemaphoreType.DMA,
            pltpu.SemaphoreType.DMA,
        ],
        compiler_params=pltpu.CompilerParams(
            use_tc_tiling_on_sc=False, needs_layout_passes=False
        ),
    )
    def k(tab_hbm, idx_hbm, out_hbm, raw_v, col_v, grp0_v, grp1_v, ob0_v,
          ob1_v, sem_g, sem_o):
        grp_b = (grp0_v, grp1_v)
        ob_b = (ob0_v, ob1_v)
        wid = lax.axis_index("s") * NC + lax.axis_index("c")
        base = wid * b_per_w
        pltpu.sync_copy(idx_hbm.at[pl.ds(base, b_per_w)], raw_v)

        iota = lax.iota(jnp.int32, L)
        iotaJ = iota * J

        def regroup(g, _):
            sl, j = lax.div(g, J), lax.rem(g, J)
            cbase = sl * (_RB * J) + j
            xs = [plsc.load_gather(raw_v, [cbase + (m * L) * J + iotaJ])
                  for m in range(_RB // L)]
            for m in range(_RB // L):
                col_v[pl.ds(g * _RB + m * L, L)] = xs[m]
            return 0

        lax.fori_loop(0, n_cells, regroup, 0)

        def gather(t, b):
            return pltpu.async_copy(
                tab_hbm.at[col_v.at[pl.ds(t * _RB, _RB)]], grp_b[b], sem_g
            )

        sks = [(iota + m) & (_D - 1) for m in range(_D)]
        dks = [
            lax.shift_right_logical(sk, 3) * (8 * _RB)
            + (sk & 7) * _RB
            + iota
            for sk in sks
        ]

        def transpose(b):
            def body(q, _):
                b0 = q * L
                row = b0 + iota
                xs = [plsc.load_gather(grp_b[b], [row, sks[m]])
                      for m in range(_D)]
                for m in range(_D):
                    plsc.store_scatter(ob_b[b], [dks[m] + b0], xs[m])
                return 0

            lax.fori_loop(0, _RB // L, body, 0)

        def put(t, b):
            sl, j = lax.div(t, J), lax.rem(t, J)
            s = wid * SPW + sl
            for fb in range(_D // 8):
                pltpu.async_copy(
                    ob_b[b].at[pl.ds(fb * (8 * _RB), 8 * _RB)],
                    out_hbm.at[
                        pl.ds(((j * (_D // 8) + fb) * (R // _RB) + s)
                              * (8 * _RB), 8 * _RB)
                    ],
                    sem_o,
                )

        def wait_puts(b):
            pltpu.make_async_copy(
                ob_b[b], out_hbm.at[pl.ds(0, _RB * _D)], sem_o
            ).wait()

        gather(0, 0)

        def cell_pair(c, _):
            for b in range(2):
                t = c * 2 + b

                @pl.when(t + 1 < n_cells)
                def _():
                    gather(t + 1, 1 - b)

                pltpu.make_async_copy(
                    tab_hbm.at[col_v.at[pl.ds(t * _RB, _RB)]], grp_b[b], sem_g
                ).wait()

                @pl.when(t >= 2)
                def _():
                    wait_puts(b)

                transpose(b)
                put(t, b)
            return 0

        lax.fori_loop(0, n_cells // 2, cell_pair, 0)
        for b in range(2):
            wait_puts(b)

    return k


def kernel(token_ids, weight):
    R, J = token_ids.shape
    V, D = weight.shape
    flat = token_ids.reshape(R * J).astype(jnp.int32)
    out = _make_gather(R, J)(weight, flat)
    o5 = out.reshape(J, D // 8, R // _RB, 8, _RB)
    return o5.transpose(2, 4, 0, 1, 3).reshape(R, J, D)


# speedup vs baseline: 2.1445x; 1.0059x over previous
import functools

import jax
import jax.numpy as jnp
from jax import lax
from jax.experimental import pallas as pl
from jax.experimental.pallas import tpu as pltpu
from jax.experimental.pallas import tpu_sc as plsc

_D = 32
_RB = 128


@functools.lru_cache(maxsize=None)
def _make_gather(R, J):
    info = plsc.get_sparse_core_info()
    NC, NS, L = info.num_cores, info.num_subcores, info.num_lanes
    NW = NC * NS
    assert R % (_RB * NW) == 0 and J <= 32
    SPW = R // (_RB * NW)
    n_cells = SPW * J
    assert n_cells % 2 == 0
    NS_ALL = R // _RB
    B = R * J
    mesh = plsc.VectorSubcoreMesh(core_axis_name="c", subcore_axis_name="s")

    @functools.partial(
        pl.kernel,
        mesh=mesh,
        out_type=jax.ShapeDtypeStruct((B * _D,), jnp.float32),
        scratch_types=[
            pltpu.VMEM((SPW * 4 * 8 * _RB,), jnp.int32),
            pltpu.VMEM((_RB, _D), jnp.float32),
            pltpu.VMEM((_RB, _D), jnp.float32),
            pltpu.VMEM((_RB * _D,), jnp.float32),
            pltpu.VMEM((_RB * _D,), jnp.float32),
            pltpu.SemaphoreType.DMA,
            pltpu.SemaphoreType.DMA,
            pltpu.SemaphoreType.DMA,
        ],
        compiler_params=pltpu.CompilerParams(
            use_tc_tiling_on_sc=False, needs_layout_passes=False
        ),
    )
    def k(tab_hbm, idx_hbm, out_hbm, idx_v, grp0_v, grp1_v, ob0_v, ob1_v,
          sem_i, sem_g, sem_o):
        grp_b = (grp0_v, grp1_v)
        ob_b = (ob0_v, ob1_v)
        wid = lax.axis_index("s") * NC + lax.axis_index("c")

        for sl in range(SPW):
            for jb in range(4):
                pltpu.async_copy(
                    idx_hbm.at[
                        pl.ds(((jb * NS_ALL + wid * SPW + sl) * 8) * _RB,
                              8 * _RB)
                    ],
                    idx_v.at[pl.ds((sl * 4 + jb) * 8 * _RB, 8 * _RB)],
                    sem_i,
                )
        for sl in range(SPW):
            for jb in range(4):
                pltpu.make_async_copy(
                    idx_hbm.at[pl.ds(0, 8 * _RB)],
                    idx_v.at[pl.ds(0, 8 * _RB)],
                    sem_i,
                ).wait()

        iota = lax.iota(jnp.int32, L)

        def col_off(t):
            sl, j = lax.div(t, J), lax.rem(t, J)
            return (sl * 4 + lax.div(j, 8)) * (8 * _RB) + lax.rem(j, 8) * _RB

        def gather(t, b):
            return pltpu.async_copy(
                tab_hbm.at[idx_v.at[pl.ds(col_off(t), _RB)]], grp_b[b], sem_g
            )

        sks = [(iota + m) & (_D - 1) for m in range(_D)]
        dks = [
            lax.shift_right_logical(sk, 3) * (8 * _RB)
            + (sk & 7) * _RB
            + iota
            for sk in sks
        ]

        def transpose(b):
            def body(q, _):
                b0 = q * L
                row = b0 + iota
                xs = [plsc.load_gather(grp_b[b], [row, sks[m]])
                      for m in range(_D)]
                for m in range(_D):
                    plsc.store_scatter(ob_b[b], [dks[m] + b0], xs[m])
                return 0

            lax.fori_loop(0, _RB // L, body, 0)

        def put(t, b):
            sl, j = lax.div(t, J), lax.rem(t, J)
            s = wid * SPW + sl
            for fb in range(_D // 8):
                pltpu.async_copy(
                    ob_b[b].at[pl.ds(fb * (8 * _RB), 8 * _RB)],
                    out_hbm.at[
                        pl.ds(((j * (_D // 8) + fb) * NS_ALL + s) * (8 * _RB),
                              8 * _RB)
                    ],
                    sem_o,
                )

        def wait_puts(b):
            pltpu.make_async_copy(
                ob_b[b], out_hbm.at[pl.ds(0, _RB * _D)], sem_o
            ).wait()

        gather(0, 0)

        def cell_pair(c, _):
            for b in range(2):
                t = c * 2 + b

                @pl.when(t + 1 < n_cells)
                def _():
                    gather(t + 1, 1 - b)

                pltpu.make_async_copy(
                    tab_hbm.at[idx_v.at[pl.ds(col_off(t), _RB)]], grp_b[b],
                    sem_g,
                ).wait()

                @pl.when(t >= 2)
                def _():
                    wait_puts(b)

                transpose(b)
                put(t, b)
            return 0

        lax.fori_loop(0, n_cells // 2, cell_pair, 0)
        for b in range(2):
            wait_puts(b)

    return k


def kernel(token_ids, weight):
    R, J = token_ids.shape
    V, D = weight.shape
    tokp = jnp.pad(token_ids.astype(jnp.int32), ((0, 0), (0, 32 - J)))
    tok4 = tokp.T.reshape(4, 8, R // _RB, _RB).transpose(0, 2, 1, 3)
    out = _make_gather(R, J)(weight, tok4.reshape(-1))
    o5 = out.reshape(J, D // 8, R // _RB, 8, _RB)
    return o5.transpose(2, 4, 0, 1, 3).reshape(R, J, D)
